# baseline (speedup 1.0000x reference)
import functools

import jax
import jax.numpy as jnp
from jax import lax
from jax.experimental import pallas as pl
from jax.experimental.pallas import tpu as pltpu
from jax.experimental.pallas import tpu_sc as plsc

NUM_EMB = 1000000
DIM = 64
NSAMP = 16384
SEQ = 50
NW = 32
SAMP_PER_W = NSAMP // NW
CHUNK = 128
NCH = SAMP_PER_W // CHUNK


@functools.partial(
    pl.kernel,
    mesh=plsc.VectorSubcoreMesh(core_axis_name="c", subcore_axis_name="s"),
    out_type=jax.ShapeDtypeStruct((SEQ, 8, NSAMP // CHUNK, 8, CHUNK),
                                  jnp.float32),
    scratch_types=[
        pltpu.VMEM((SAMP_PER_W, SEQ), jnp.int32),
        pltpu.VMEM((SEQ, SAMP_PER_W), jnp.int32),
        pltpu.VMEM((NCH, CHUNK, DIM), jnp.float32),
        pltpu.VMEM((NCH, 8, 8, CHUNK), jnp.float32),
        pltpu.SemaphoreType.DMA,
        pltpu.SemaphoreType.DMA,
    ],
    compiler_params=pltpu.CompilerParams(
        use_tc_tiling_on_sc=False, needs_layout_passes=False),
)
def _gather(idx_hbm, table_hbm, out_hbm, idx_v, idx_t, rows_v, trans_v,
            sem_g, sem_w):
    wid = lax.axis_index("s") * 2 + lax.axis_index("c")
    b0 = wid * SAMP_PER_W

    pltpu.sync_copy(idx_hbm.at[pl.ds(b0, SAMP_PER_W)], idx_v)
    iota = lax.iota(jnp.int32, 16)

    def tr_idx(s, carry):
        for j in range(SAMP_PER_W // 16):
            v = plsc.load_gather(
                idx_v, [iota + 16 * j, jnp.full((16,), s, jnp.int32)])
            idx_t[s, pl.ds(16 * j, 16)] = v
        return carry

    lax.fori_loop(0, SEQ, tr_idx, 0)

    def fire(s, b):
        pltpu.make_async_copy(
            table_hbm.at[idx_t.at[s, pl.ds(b * CHUNK, CHUNK)]],
            rows_v.at[b], sem_g).start()

    def wait_g(s, b):
        pltpu.make_async_copy(
            table_hbm.at[idx_t.at[s, pl.ds(b * CHUNK, CHUNK)]],
            rows_v.at[b], sem_g).wait()

    def transpose(b):
        rows = rows_v.at[b]
        row_vecs = [iota + 16 * j for j in range(CHUNK // 16)]

        def tbody(c_hi, carry):
            for c_lo in range(8):
                col = jnp.full((16,), 8 * c_hi + c_lo, jnp.int32)
                for j in range(CHUNK // 16):
                    v = plsc.load_gather(rows, [row_vecs[j], col])
                    trans_v[b, c_hi, c_lo, pl.ds(16 * j, 16)] = v
            return carry

        lax.fori_loop(0, 8, tbody, 0)

    def write(s, b):
        return pltpu.make_async_copy(
            trans_v.at[b], out_hbm.at[s, :, wid * NCH + b], sem_w)

    for b in range(NCH):
        fire(0, b)

    def body(t, carry):
        for b in range(NCH):
            wait_g(t, b)

            @pl.when(t > 0)
            def _():
                write(t - 1, b).wait()

            transpose(b)
            write(t, b).start()
            fire(t + 1, b)
        return carry

    lax.fori_loop(0, SEQ - 1, body, 0)

    for b in range(NCH):
        wait_g(SEQ - 1, b)
        write(SEQ - 2, b).wait()
        transpose(b)
        write(SEQ - 1, b).start()
    for b in range(NCH):
        write(SEQ - 1, b).wait()


def kernel(input_, weight):
    out5 = _gather(input_.astype(jnp.int32), weight)
    return out5.transpose((2, 4, 0, 1, 3)).reshape(NSAMP, SEQ, DIM)


# speedup vs baseline: 1.8285x; 1.8285x over previous
import functools

import jax
import jax.numpy as jnp
from jax import lax
from jax.experimental import pallas as pl
from jax.experimental.pallas import tpu as pltpu
from jax.experimental.pallas import tpu_sc as plsc

NUM_EMB = 1000000
DIM = 64
NSAMP = 16384
SEQ = 50
NW = 32
SAMP_PER_W = NSAMP // NW
CHUNK = 128
NCH = SAMP_PER_W // CHUNK


@functools.partial(
    pl.kernel,
    mesh=plsc.VectorSubcoreMesh(core_axis_name="c", subcore_axis_name="s"),
    out_type=jax.ShapeDtypeStruct((SEQ, 8, NSAMP // CHUNK, 8, CHUNK),
                                  jnp.float32),
    scratch_types=[
        pltpu.VMEM((SAMP_PER_W, SEQ), jnp.int32),
        pltpu.VMEM((SEQ, SAMP_PER_W), jnp.int32),
        pltpu.VMEM((NCH, CHUNK, DIM), jnp.float32),
        pltpu.VMEM((NCH, 8, 8, CHUNK + 1), jnp.float32),
        pltpu.SemaphoreType.DMA,
        pltpu.SemaphoreType.DMA,
    ],
    compiler_params=pltpu.CompilerParams(
        use_tc_tiling_on_sc=False, needs_layout_passes=False),
)
def _gather(idx_hbm, table_hbm, out_hbm, idx_v, idx_t, rows_v, trans_v,
            sem_g, sem_w):
    wid = lax.axis_index("s") * 2 + lax.axis_index("c")
    b0 = wid * SAMP_PER_W

    pltpu.sync_copy(idx_hbm.at[pl.ds(b0, SAMP_PER_W)], idx_v)
    iota = lax.iota(jnp.int32, 16)

    def tr_idx(s, carry):
        for j in range(SAMP_PER_W // 16):
            v = plsc.load_gather(
                idx_v, [iota + 16 * j, jnp.full((16,), s, jnp.int32)])
            idx_t[s, pl.ds(16 * j, 16)] = v
        return carry

    lax.fori_loop(0, SEQ, tr_idx, 0)

    def fire(s, b):
        pltpu.make_async_copy(
            table_hbm.at[idx_t.at[s, pl.ds(b * CHUNK, CHUNK)]],
            rows_v.at[b], sem_g).start()

    def wait_g(s, b):
        pltpu.make_async_copy(
            table_hbm.at[idx_t.at[s, pl.ds(b * CHUNK, CHUNK)]],
            rows_v.at[b], sem_g).wait()

    def transpose(b):
        tr = trans_v.at[b]
        chs = [
            (lax.shift_right_logical(iota + 16 * j, 3), (iota + 16 * j) & 7)
            for j in range(DIM // 16)
        ]

        def tbody(bb, carry):
            bvec = jnp.full((16,), bb, jnp.int32)
            for j in range(DIM // 16):
                v = rows_v[b, bb, pl.ds(16 * j, 16)]
                plsc.store_scatter(tr, [chs[j][0], chs[j][1], bvec], v)
            return carry

        lax.fori_loop(0, CHUNK, tbody, 0)

    def write(s, b):
        return pltpu.make_async_copy(
            trans_v.at[b, :, :, pl.ds(0, CHUNK)],
            out_hbm.at[s, :, wid * NCH + b], sem_w)

    for b in range(NCH):
        fire(0, b)

    def body(t, carry):
        for b in range(NCH):
            wait_g(t, b)

            @pl.when(t > 0)
            def _():
                write(t - 1, b).wait()

            transpose(b)
            write(t, b).start()
            fire(t + 1, b)
        return carry

    lax.fori_loop(0, SEQ - 1, body, 0)

    for b in range(NCH):
        wait_g(SEQ - 1, b)
        write(SEQ - 2, b).wait()
        transpose(b)
        write(SEQ - 1, b).start()
    for b in range(NCH):
        write(SEQ - 1, b).wait()


def kernel(input_, weight):
    out5 = _gather(input_.astype(jnp.int32), weight)
    return out5.transpose((2, 4, 0, 1, 3)).reshape(NSAMP, SEQ, DIM)


# speedup vs baseline: 1.8705x; 1.0229x over previous
import functools

import jax
import jax.numpy as jnp
from jax import lax
from jax.experimental import pallas as pl
from jax.experimental.pallas import tpu as pltpu
from jax.experimental.pallas import tpu_sc as plsc

NUM_EMB = 1000000
DIM = 64
NSAMP = 16384
SEQ = 50
NW = 32
SAMP_PER_W = NSAMP // NW
CHUNK = 128
NCH = SAMP_PER_W // CHUNK


@functools.partial(
    pl.kernel,
    mesh=plsc.VectorSubcoreMesh(core_axis_name="c", subcore_axis_name="s"),
    out_type=jax.ShapeDtypeStruct((SEQ, 8, NSAMP // CHUNK, 8, CHUNK),
                                  jnp.float32),
    scratch_types=[
        pltpu.VMEM((SAMP_PER_W, SEQ), jnp.int32),
        pltpu.VMEM((SEQ, SAMP_PER_W), jnp.int32),
        pltpu.VMEM((NCH, CHUNK, DIM), jnp.float32),
        pltpu.VMEM((NCH, 8, 8, CHUNK + 1), jnp.float32),
        pltpu.SemaphoreType.DMA,
        pltpu.SemaphoreType.DMA,
    ],
    compiler_params=pltpu.CompilerParams(
        use_tc_tiling_on_sc=False, needs_layout_passes=False),
)
def _gather(idx_hbm, table_hbm, out_hbm, idx_v, idx_t, rows_v, trans_v,
            sem_g, sem_w):
    wid = lax.axis_index("s") * 2 + lax.axis_index("c")
    b0 = wid * SAMP_PER_W

    pltpu.sync_copy(idx_hbm.at[pl.ds(b0, SAMP_PER_W)], idx_v)
    iota = lax.iota(jnp.int32, 16)

    def tr_idx(s, carry):
        for j in range(SAMP_PER_W // 16):
            v = plsc.load_gather(
                idx_v, [iota + 16 * j, jnp.full((16,), s, jnp.int32)])
            idx_t[s, pl.ds(16 * j, 16)] = v
        return carry

    lax.fori_loop(0, SEQ, tr_idx, 0)

    def fire(s, b):
        pltpu.make_async_copy(
            table_hbm.at[idx_t.at[s, pl.ds(b * CHUNK, CHUNK)]],
            rows_v.at[b], sem_g).start()

    def wait_g(s, b):
        pltpu.make_async_copy(
            table_hbm.at[idx_t.at[s, pl.ds(b * CHUNK, CHUNK)]],
            rows_v.at[b], sem_g).wait()

    def transpose(b):
        tr = trans_v.at[b]
        chs = [
            (lax.shift_right_logical(iota + 16 * j, 3), (iota + 16 * j) & 7)
            for j in range(DIM // 16)
        ]

        def tbody(q, carry):
            bb0 = q * 4
            bvec = jnp.full((16,), bb0, jnp.int32)
            for r in range(4):
                for j in range(DIM // 16):
                    v = rows_v[b, bb0 + r, pl.ds(16 * j, 16)]
                    plsc.store_scatter(
                        tr, [chs[j][0], chs[j][1], bvec + r], v)
            return carry

        lax.fori_loop(0, CHUNK // 4, tbody, 0)

    def write(s, b):
        return pltpu.make_async_copy(
            trans_v.at[b, :, :, pl.ds(0, CHUNK)],
            out_hbm.at[s, :, wid * NCH + b], sem_w)

    for b in range(NCH):
        fire(0, b)

    def body(t, carry):
        for b in range(NCH):
            wait_g(t, b)

            @pl.when(t > 0)
            def _():
                write(t - 1, b).wait()

            transpose(b)
            write(t, b).start()
            fire(t + 1, b)
        return carry

    lax.fori_loop(0, SEQ - 1, body, 0)

    for b in range(NCH):
        wait_g(SEQ - 1, b)
        write(SEQ - 2, b).wait()
        transpose(b)
        write(SEQ - 1, b).start()
    for b in range(NCH):
        write(SEQ - 1, b).wait()


def kernel(input_, weight):
    out5 = _gather(input_.astype(jnp.int32), weight)
    return out5.transpose((2, 4, 0, 1, 3)).reshape(NSAMP, SEQ, DIM)


# speedup vs baseline: 1.8758x; 1.0029x over previous
import functools

import jax
import jax.numpy as jnp
from jax import lax
from jax.experimental import pallas as pl
from jax.experimental.pallas import tpu as pltpu
from jax.experimental.pallas import tpu_sc as plsc

NUM_EMB = 1000000
DIM = 64
NSAMP = 16384
SEQ = 50
NW = 32
SAMP_PER_W = NSAMP // NW
CHUNK = 128
NCH = SAMP_PER_W // CHUNK


@functools.partial(
    pl.kernel,
    mesh=plsc.VectorSubcoreMesh(core_axis_name="c", subcore_axis_name="s"),
    out_type=jax.ShapeDtypeStruct((SEQ, 8, NSAMP // CHUNK, 8, CHUNK),
                                  jnp.float32),
    scratch_types=[
        pltpu.VMEM((SAMP_PER_W, SEQ), jnp.int32),
        pltpu.VMEM((SEQ, SAMP_PER_W), jnp.int32),
        pltpu.VMEM((NCH, CHUNK, DIM), jnp.float32),
        pltpu.VMEM((NCH, 8, 8, CHUNK + 1), jnp.float32),
        pltpu.SemaphoreType.DMA,
        pltpu.SemaphoreType.DMA,
    ],
    compiler_params=pltpu.CompilerParams(
        use_tc_tiling_on_sc=False, needs_layout_passes=False),
)
def _gather(idx_hbm, table_hbm, out_hbm, idx_v, idx_t, rows_v, trans_v,
            sem_g, sem_w):
    wid = lax.axis_index("s") * 2 + lax.axis_index("c")
    b0 = wid * SAMP_PER_W

    pltpu.sync_copy(idx_hbm.at[pl.ds(b0, SAMP_PER_W)], idx_v)
    iota = lax.iota(jnp.int32, 16)

    def tr_idx(s, carry):
        for j in range(SAMP_PER_W // 16):
            v = plsc.load_gather(
                idx_v, [iota + 16 * j, jnp.full((16,), s, jnp.int32)])
            idx_t[s, pl.ds(16 * j, 16)] = v
        return carry

    lax.fori_loop(0, SEQ, tr_idx, 0)

    def fire(s, b):
        pltpu.make_async_copy(
            table_hbm.at[idx_t.at[s, pl.ds(b * CHUNK, CHUNK)]],
            rows_v.at[b], sem_g).start()

    def wait_g(s, b):
        pltpu.make_async_copy(
            table_hbm.at[idx_t.at[s, pl.ds(b * CHUNK, CHUNK)]],
            rows_v.at[b], sem_g).wait()

    def transpose(b):
        tr = trans_v.at[b]
        chs = [
            (lax.shift_right_logical(iota + 16 * j, 3), (iota + 16 * j) & 7)
            for j in range(DIM // 16)
        ]

        def tbody(q, carry):
            bb0 = q * 8
            bvec = jnp.full((16,), bb0, jnp.int32)
            for r in range(8):
                for j in range(DIM // 16):
                    v = rows_v[b, bb0 + r, pl.ds(16 * j, 16)]
                    plsc.store_scatter(
                        tr, [chs[j][0], chs[j][1], bvec + r], v)
            return carry

        lax.fori_loop(0, CHUNK // 8, tbody, 0)

    def write(s, b):
        return pltpu.make_async_copy(
            trans_v.at[b, :, :, pl.ds(0, CHUNK)],
            out_hbm.at[s, :, wid * NCH + b], sem_w)

    for b in range(NCH):
        fire(0, b)

    def body(t, carry):
        for b in range(NCH):
            wait_g(t, b)

            @pl.when(t > 0)
            def _():
                write(t - 1, b).wait()

            transpose(b)
            write(t, b).start()
            fire(t + 1, b)
        return carry

    lax.fori_loop(0, SEQ - 1, body, 0)

    for b in range(NCH):
        wait_g(SEQ - 1, b)
        write(SEQ - 2, b).wait()
        transpose(b)
        write(SEQ - 1, b).start()
    for b in range(NCH):
        write(SEQ - 1, b).wait()


def kernel(input_, weight):
    out5 = _gather(input_.astype(jnp.int32), weight)
    return out5.transpose((2, 4, 0, 1, 3)).reshape(NSAMP, SEQ, DIM)


# speedup vs baseline: 2.1837x; 1.1641x over previous
import functools

import jax
import jax.numpy as jnp
from jax import lax
from jax.experimental import pallas as pl
from jax.experimental.pallas import tpu as pltpu
from jax.experimental.pallas import tpu_sc as plsc

NUM_EMB = 1000000
DIM = 64
NSAMP = 16384
SEQ = 50
NW = 32
SAMP_PER_W = NSAMP // NW
CHUNK = 128
NCH = SAMP_PER_W // CHUNK


@functools.partial(
    pl.kernel,
    mesh=plsc.VectorSubcoreMesh(core_axis_name="c", subcore_axis_name="s"),
    out_type=jax.ShapeDtypeStruct((SEQ, 8, NSAMP // CHUNK, 8, CHUNK),
                                  jnp.float32),
    scratch_types=[
        pltpu.VMEM((SAMP_PER_W, SEQ), jnp.int32),
        pltpu.VMEM((SEQ, SAMP_PER_W), jnp.int32),
        pltpu.VMEM((NCH, CHUNK, DIM), jnp.float32),
        pltpu.VMEM((NCH, 8, 8, CHUNK + 1), jnp.float32),
        pltpu.SemaphoreType.DMA,
        pltpu.SemaphoreType.DMA,
    ],
    compiler_params=pltpu.CompilerParams(
        use_tc_tiling_on_sc=False, needs_layout_passes=False),
)
def _gather(idx_hbm, table_hbm, out_hbm, idx_v, idx_t, rows_v, trans_v,
            sem_g, sem_w):
    wid = lax.axis_index("s") * 2 + lax.axis_index("c")
    b0 = wid * SAMP_PER_W

    pltpu.sync_copy(idx_hbm.at[pl.ds(b0, SAMP_PER_W)], idx_v)
    iota = lax.iota(jnp.int32, 16)

    def tr_idx(s, carry):
        for j in range(SAMP_PER_W // 16):
            v = plsc.load_gather(
                idx_v, [iota + 16 * j, jnp.full((16,), s, jnp.int32)])
            idx_t[s, pl.ds(16 * j, 16)] = v
        return carry

    lax.fori_loop(0, SEQ, tr_idx, 0)

    def fire(s, b):
        pltpu.make_async_copy(
            table_hbm.at[idx_t.at[s, pl.ds(b * CHUNK, CHUNK)]],
            rows_v.at[b], sem_g).start()

    def wait_g(s, b):
        pltpu.make_async_copy(
            table_hbm.at[idx_t.at[s, pl.ds(b * CHUNK, CHUNK)]],
            rows_v.at[b], sem_g).wait()

    def transpose(b):
        tr = trans_v.at[b]
        chs = [
            (lax.shift_right_logical(iota + 16 * j, 3), (iota + 16 * j) & 7)
            for j in range(DIM // 16)
        ]

        @plsc.parallel_loop(0, CHUNK // 8, unroll=2)
        def tbody(q):
            bb0 = q * 8
            bvec = jnp.full((16,), bb0, jnp.int32)
            for r in range(8):
                for j in range(DIM // 16):
                    v = rows_v[b, bb0 + r, pl.ds(16 * j, 16)]
                    plsc.store_scatter(
                        tr, [chs[j][0], chs[j][1], bvec + r], v)

    def write(s, b):
        return pltpu.make_async_copy(
            trans_v.at[b, :, :, pl.ds(0, CHUNK)],
            out_hbm.at[s, :, wid * NCH + b], sem_w)

    for b in range(NCH):
        fire(0, b)

    def body(t, carry):
        for b in range(NCH):
            wait_g(t, b)

            @pl.when(t > 0)
            def _():
                write(t - 1, b).wait()

            transpose(b)
            write(t, b).start()
            fire(t + 1, b)
        return carry

    lax.fori_loop(0, SEQ - 1, body, 0)

    for b in range(NCH):
        wait_g(SEQ - 1, b)
        write(SEQ - 2, b).wait()
        transpose(b)
        write(SEQ - 1, b).start()
    for b in range(NCH):
        write(SEQ - 1, b).wait()


def kernel(input_, weight):
    out5 = _gather(input_.astype(jnp.int32), weight)
    return out5.transpose((2, 4, 0, 1, 3)).reshape(NSAMP, SEQ, DIM)
